# baseline (speedup 1.0000x reference)
import functools

import jax
import jax.numpy as jnp
from jax.experimental import pallas as pl
from jax.experimental.pallas import tpu as pltpu

_KNUM_POINTS = [1024, 256]
_KNUM_SAMPLE = [32, 64]


def _fps_body(x_ref, y_ref, z_ref, xt_ref, fid_ref, dist_ref, *, npoint):
    B, N = x_ref.shape
    dist_ref[...] = jnp.full((B, N), 1e10, jnp.float32)
    iota = jax.lax.broadcasted_iota(jnp.int32, (B, N), 1)

    def step(i, far):
        fid_ref[pl.ds(i, 1)] = far.reshape(1, 1, B)
        cx, cy, cz = [], [], []
        for b in range(B):
            fb = far[0, b]
            row = xt_ref[pl.ds(fb, 1)]
            cx.append(row[0, 0, 3 * b + 0])
            cy.append(row[0, 0, 3 * b + 1])
            cz.append(row[0, 0, 3 * b + 2])
        cxv = jnp.stack(cx).reshape(B, 1)
        cyv = jnp.stack(cy).reshape(B, 1)
        czv = jnp.stack(cz).reshape(B, 1)
        dx = x_ref[...] - cxv
        dy = y_ref[...] - cyv
        dz = z_ref[...] - czv
        d = (dx * dx + dy * dy) + dz * dz
        dist = jnp.minimum(dist_ref[...], d)
        dist_ref[...] = dist
        m = jnp.max(dist, axis=-1, keepdims=True)
        nxt = jnp.min(jnp.where(dist == m, iota, N), axis=-1, keepdims=True)
        return nxt.astype(jnp.int32).reshape(1, B)

    jax.lax.fori_loop(0, npoint, step, jnp.zeros((1, B), jnp.int32))


def _fps_pallas(points, npoint):
    B, _, N = points.shape
    x = points[:, 0, :]
    y = points[:, 1, :]
    z = points[:, 2, :]
    xt = jnp.transpose(points, (2, 0, 1)).reshape(N, 1, 3 * B)
    fid = pl.pallas_call(
        functools.partial(_fps_body, npoint=npoint),
        out_shape=jax.ShapeDtypeStruct((npoint, 1, B), jnp.int32),
        scratch_shapes=[pltpu.VMEM((B, N), jnp.float32)],
    )(x, y, z, xt)
    return jnp.transpose(fid[:, 0, :], (1, 0))


def _gather(points, idx):
    pt = jnp.transpose(points, (0, 2, 1))
    out = jnp.take_along_axis(pt, idx[:, :, None], axis=1)
    return jnp.transpose(out, (0, 2, 1))


def _knn(k, xyz, new_xyz):
    d = (jnp.sum(new_xyz ** 2, axis=-1)[:, :, None]
         + jnp.sum(xyz ** 2, axis=-1)[:, None, :]
         - 2.0 * jnp.einsum('bsd,bnd->bsn', new_xyz, xyz))
    _, idx = jax.lax.top_k(-d, k)
    return idx


def _group(feats, idx):
    B, C, N = feats.shape
    _, S, K = idx.shape
    ft = jnp.transpose(feats, (0, 2, 1)).reshape(B * N, C)
    flat = (idx.reshape(B, S * K) + (jnp.arange(B, dtype=jnp.int32) * N)[:, None]).reshape(-1)
    g = jnp.take(ft, flat, axis=0)
    return jnp.transpose(g.reshape(B, S, K, C), (0, 3, 1, 2))


def _conv_bn_relu(x, W, b, g, be):
    y = jnp.einsum('oc,bcsk->bosk', W, x) + b[None, :, None, None]
    mean = jnp.mean(y, axis=(0, 2, 3), keepdims=True)
    var = jnp.var(y, axis=(0, 2, 3), keepdims=True)
    y = (y - mean) / jnp.sqrt(var + 1e-5) * g[None, :, None, None] + be[None, :, None, None]
    return jax.nn.relu(y)


def _copy_kernel(x_ref, o_ref):
    o_ref[...] = x_ref[...]


def _pl_copy(x):
    return pl.pallas_call(
        _copy_kernel,
        out_shape=jax.ShapeDtypeStruct(x.shape, x.dtype),
    )(x)


def kernel(points,
           W_l1c0, b_l1c0, g_l1c0, be_l1c0,
           W_l1c1, b_l1c1, g_l1c1, be_l1c1,
           W_l1c2, b_l1c2, g_l1c2, be_l1c2,
           W_l2c0, b_l2c0, g_l2c0, be_l2c0,
           W_l2c1, b_l2c1, g_l2c1, be_l2c1,
           W_l2c2, b_l2c2, g_l2c2, be_l2c2):
    kw = locals()
    names = ["l1c0", "l1c1", "l1c2", "l2c0", "l2c1", "l2c2"]
    params = [(kw["W_" + n], kw["b_" + n], kw["g_" + n], kw["be_" + n]) for n in names]

    feats = points
    points_list, feats_list, gidx_list = [], [], []
    offs = [0, 3]
    for li in range(2):
        npoint = _KNUM_POINTS[li]
        nsample = _KNUM_SAMPLE[li]
        xyz = jnp.transpose(points, (0, 2, 1))
        fid = _fps_pallas(points, npoint)
        prop = _gather(points, fid)
        new_xyz = jnp.transpose(prop, (0, 2, 1))
        gidx = _knn(nsample, jax.lax.stop_gradient(xyz), jax.lax.stop_gradient(new_xyz))
        gp = _group(points, gidx)
        gpn = gp - prop[..., None]
        gf = gp if li == 0 else _group(feats, gidx)
        x = jnp.concatenate([gpn, gf], axis=1)
        for n in range(3):
            W, b, g, be = params[offs[li] + n]
            x = _conv_bn_relu(x, W, b, g, be)
        pf = jnp.max(x, axis=-1)
        points_list.append(prop)
        feats_list.append(pf)
        gidx_list.append(gidx)
        points = prop
        feats = pf
    return (*points_list, *feats_list, *gidx_list)


# speedup vs baseline: 1.3378x; 1.3378x over previous
import functools

import jax
import jax.numpy as jnp
from jax.experimental import pallas as pl
from jax.experimental.pallas import tpu as pltpu

_KNUM_POINTS = [1024, 256]
_KNUM_SAMPLE = [32, 64]


def _fps_body(x_ref, y_ref, z_ref, xt_ref, fid_ref, dist_ref, *, npoint):
    B, N = x_ref.shape
    dist_ref[...] = jnp.full((B, N), 1e10, jnp.float32)
    iota = jax.lax.broadcasted_iota(jnp.int32, (B, N), 1)

    def step(i, far):
        fid_ref[pl.ds(i, 1)] = far.reshape(1, 1, B)
        cx, cy, cz = [], [], []
        for b in range(B):
            fb = far[0, b]
            row = xt_ref[pl.ds(fb, 1)]
            cx.append(row[0, 0, 3 * b + 0])
            cy.append(row[0, 0, 3 * b + 1])
            cz.append(row[0, 0, 3 * b + 2])
        cxv = jnp.stack(cx).reshape(B, 1)
        cyv = jnp.stack(cy).reshape(B, 1)
        czv = jnp.stack(cz).reshape(B, 1)
        dx = x_ref[...] - cxv
        dy = y_ref[...] - cyv
        dz = z_ref[...] - czv
        d = (dx * dx + dy * dy) + dz * dz
        dist = jnp.minimum(dist_ref[...], d)
        dist_ref[...] = dist
        m = jnp.max(dist, axis=-1, keepdims=True)
        nxt = jnp.min(jnp.where(dist == m, iota, N), axis=-1, keepdims=True)
        return nxt.astype(jnp.int32).reshape(1, B)

    jax.lax.fori_loop(0, npoint, step, jnp.zeros((1, B), jnp.int32))


def _fps_pallas(points, npoint):
    B, _, N = points.shape
    x = points[:, 0, :]
    y = points[:, 1, :]
    z = points[:, 2, :]
    xt = jnp.transpose(points, (2, 0, 1)).reshape(N, 1, 3 * B)
    fid = pl.pallas_call(
        functools.partial(_fps_body, npoint=npoint),
        out_shape=jax.ShapeDtypeStruct((npoint, 1, B), jnp.int32),
        scratch_shapes=[pltpu.VMEM((B, N), jnp.float32)],
    )(x, y, z, xt)
    return jnp.transpose(fid[:, 0, :], (1, 0))


def _gather(points, idx):
    pt = jnp.transpose(points, (0, 2, 1))
    out = jnp.take_along_axis(pt, idx[:, :, None], axis=1)
    return jnp.transpose(out, (0, 2, 1))


def _knn_body(q_ref, pt_ref, qn2_ref, pn2_ref, out_ref, *, K, N):
    _INF = jnp.float32(jnp.inf)
    _BIGN = jnp.int32(2**31 - 1)
    NR = N // 128
    q = q_ref[0]
    pt = pt_ref[0]
    qn2 = qn2_ref[0]
    pn2 = pn2_ref[0]
    dot = jax.lax.dot_general(q, pt, (((1,), (0,)), ((), ())),
                              preferred_element_type=jnp.float32)
    d = (qn2 + pn2) - 2.0 * dot

    lane = jax.lax.broadcasted_iota(jnp.int32, (8, 128), 1)
    cols = [d[:, c * 128:(c + 1) * 128] for c in range(NR)]
    ncols = [lane + jnp.int32(c * 128) for c in range(NR)]

    DEPTH = 4
    work = list(cols)
    sv, sn = [], []
    for j in range(DEPTH):
        mv = work[0]
        mc = jnp.zeros((8, 128), jnp.int32)
        for c in range(1, NR):
            lt = work[c] < mv
            mv = jnp.where(lt, work[c], mv)
            mc = jnp.where(lt, jnp.int32(c), mc)
        sv.append(mv)
        sn.append(mc * 128 + lane)
        if j < DEPTH - 1:
            for c in range(NR):
                work[c] = jnp.where(mc == c, _INF, work[c])

    v0, v1, v2, v3 = sv
    n0, n1, n2, n3 = sn
    outs = []
    last_v = None
    last_n = None
    for k in range(K):
        m = jnp.min(v0, axis=1, keepdims=True)
        nb = jnp.min(jnp.where(v0 == m, n0, _BIGN), axis=1, keepdims=True)
        outs.append(nb)
        last_v, last_n = m, nb
        pop = n0 == nb
        v0 = jnp.where(pop, v1, v0)
        n0 = jnp.where(pop, n1, n0)
        v1 = jnp.where(pop, v2, v1)
        n1 = jnp.where(pop, n2, n1)
        v2 = jnp.where(pop, v3, v2)
        n2 = jnp.where(pop, n3, n2)
        v3 = jnp.where(pop, _INF, v3)
        n3 = jnp.where(pop, _BIGN, n3)
    fast = jnp.concatenate(outs, axis=1)

    cnt = jnp.zeros((8, 128), jnp.int32)
    for c in range(NR):
        less = (cols[c] < last_v) | ((cols[c] == last_v) & (ncols[c] < last_n))
        cnt = cnt + jnp.where(less, jnp.int32(1), jnp.int32(0))
    total = jnp.sum(cnt, axis=1, keepdims=True)
    good = jnp.all(total == jnp.int32(K - 1))

    def exact():
        ik = jax.lax.broadcasted_iota(jnp.int32, (8, K), 1)

        def body(k, carry):
            outs_acc = carry[0]
            w = list(carry[1:])
            mv = w[0]
            mn = ncols[0]
            for c in range(1, NR):
                lt = (w[c] < mv) | ((w[c] == mv) & (ncols[c] < mn))
                mv = jnp.where(lt, w[c], mv)
                mn = jnp.where(lt, ncols[c], mn)
            m = jnp.min(mv, axis=1, keepdims=True)
            nb = jnp.min(jnp.where(mv == m, mn, _BIGN), axis=1, keepdims=True)
            outs_acc = outs_acc + jnp.where(ik == k, nb, jnp.int32(0))
            neww = [jnp.where(ncols[c] == nb, _INF, w[c]) for c in range(NR)]
            return (outs_acc, *neww)

        init = (jnp.zeros((8, K), jnp.int32), *cols)
        return jax.lax.fori_loop(0, K, body, init)[0]

    out_ref[0] = jax.lax.cond(good, lambda: fast, exact)


def _knn_pallas(xyz_pl, new_xyz, nsample):
    B, _, N = xyz_pl.shape
    S = new_xyz.shape[1]
    K = nsample
    PT = jnp.concatenate([xyz_pl, jnp.zeros((B, 5, N), jnp.float32)], axis=1)
    Q = jnp.concatenate([new_xyz, jnp.zeros((B, S, 5), jnp.float32)], axis=-1)
    qn2 = jnp.sum(new_xyz ** 2, axis=-1)[..., None]
    pn2 = jnp.sum(jnp.transpose(xyz_pl, (0, 2, 1)) ** 2, axis=-1)[:, None, :]
    grid = (B, S // 8)
    return pl.pallas_call(
        functools.partial(_knn_body, K=K, N=N),
        grid=grid,
        in_specs=[
            pl.BlockSpec((1, 8, 8), lambda b, g: (b, g, 0)),
            pl.BlockSpec((1, 8, N), lambda b, g: (b, 0, 0)),
            pl.BlockSpec((1, 8, 1), lambda b, g: (b, g, 0)),
            pl.BlockSpec((1, 1, N), lambda b, g: (b, 0, 0)),
        ],
        out_specs=pl.BlockSpec((1, 8, K), lambda b, g: (b, g, 0)),
        out_shape=jax.ShapeDtypeStruct((B, S, K), jnp.int32),
    )(Q, PT, qn2, pn2)


def _knn(k, xyz, new_xyz):
    d = (jnp.sum(new_xyz ** 2, axis=-1)[:, :, None]
         + jnp.sum(xyz ** 2, axis=-1)[:, None, :]
         - 2.0 * jnp.einsum('bsd,bnd->bsn', new_xyz, xyz))
    _, idx = jax.lax.top_k(-d, k)
    return idx


def _group(feats, idx):
    B, C, N = feats.shape
    _, S, K = idx.shape
    ft = jnp.transpose(feats, (0, 2, 1)).reshape(B * N, C)
    flat = (idx.reshape(B, S * K) + (jnp.arange(B, dtype=jnp.int32) * N)[:, None]).reshape(-1)
    g = jnp.take(ft, flat, axis=0)
    return jnp.transpose(g.reshape(B, S, K, C), (0, 3, 1, 2))


def _conv_bn_relu(x, W, b, g, be):
    y = jnp.einsum('oc,bcsk->bosk', W, x) + b[None, :, None, None]
    mean = jnp.mean(y, axis=(0, 2, 3), keepdims=True)
    var = jnp.var(y, axis=(0, 2, 3), keepdims=True)
    y = (y - mean) / jnp.sqrt(var + 1e-5) * g[None, :, None, None] + be[None, :, None, None]
    return jax.nn.relu(y)


def _copy_kernel(x_ref, o_ref):
    o_ref[...] = x_ref[...]


def _pl_copy(x):
    return pl.pallas_call(
        _copy_kernel,
        out_shape=jax.ShapeDtypeStruct(x.shape, x.dtype),
    )(x)


def kernel(points,
           W_l1c0, b_l1c0, g_l1c0, be_l1c0,
           W_l1c1, b_l1c1, g_l1c1, be_l1c1,
           W_l1c2, b_l1c2, g_l1c2, be_l1c2,
           W_l2c0, b_l2c0, g_l2c0, be_l2c0,
           W_l2c1, b_l2c1, g_l2c1, be_l2c1,
           W_l2c2, b_l2c2, g_l2c2, be_l2c2):
    kw = locals()
    names = ["l1c0", "l1c1", "l1c2", "l2c0", "l2c1", "l2c2"]
    params = [(kw["W_" + n], kw["b_" + n], kw["g_" + n], kw["be_" + n]) for n in names]

    feats = points
    points_list, feats_list, gidx_list = [], [], []
    offs = [0, 3]
    for li in range(2):
        npoint = _KNUM_POINTS[li]
        nsample = _KNUM_SAMPLE[li]
        xyz = jnp.transpose(points, (0, 2, 1))
        fid = _fps_pallas(points, npoint)
        prop = _gather(points, fid)
        new_xyz = jnp.transpose(prop, (0, 2, 1))
        gidx = _knn_pallas(points, new_xyz, nsample)
        gp = _group(points, gidx)
        gpn = gp - prop[..., None]
        gf = gp if li == 0 else _group(feats, gidx)
        x = jnp.concatenate([gpn, gf], axis=1)
        for n in range(3):
            W, b, g, be = params[offs[li] + n]
            x = _conv_bn_relu(x, W, b, g, be)
        pf = jnp.max(x, axis=-1)
        points_list.append(prop)
        feats_list.append(pf)
        gidx_list.append(gidx)
        points = prop
        feats = pf
    return (*points_list, *feats_list, *gidx_list)


# speedup vs baseline: 1.3463x; 1.0064x over previous
import functools

import jax
import jax.numpy as jnp
from jax.experimental import pallas as pl
from jax.experimental.pallas import tpu as pltpu

_KNUM_POINTS = [1024, 256]
_KNUM_SAMPLE = [32, 64]


def _fps_body(x_ref, y_ref, z_ref, xt_ref, fid_ref, dist_ref, *, npoint):
    B, N = x_ref.shape
    dist_ref[...] = jnp.full((B, N), 1e10, jnp.float32)
    iota = jax.lax.broadcasted_iota(jnp.int32, (B, N), 1)

    def step(i, far):
        fid_ref[pl.ds(i, 1)] = far.reshape(1, 1, B)
        cx, cy, cz = [], [], []
        for b in range(B):
            fb = far[0, b]
            row = xt_ref[pl.ds(fb, 1)]
            cx.append(row[0, 0, 3 * b + 0])
            cy.append(row[0, 0, 3 * b + 1])
            cz.append(row[0, 0, 3 * b + 2])
        cxv = jnp.stack(cx).reshape(B, 1)
        cyv = jnp.stack(cy).reshape(B, 1)
        czv = jnp.stack(cz).reshape(B, 1)
        dx = x_ref[...] - cxv
        dy = y_ref[...] - cyv
        dz = z_ref[...] - czv
        d = (dx * dx + dy * dy) + dz * dz
        dist = jnp.minimum(dist_ref[...], d)
        dist_ref[...] = dist
        m = jnp.max(dist, axis=-1, keepdims=True)
        nxt = jnp.min(jnp.where(dist == m, iota, N), axis=-1, keepdims=True)
        return nxt.astype(jnp.int32).reshape(1, B)

    jax.lax.fori_loop(0, npoint, step, jnp.zeros((1, B), jnp.int32))


def _fps_pallas(points, npoint):
    B, _, N = points.shape
    x = points[:, 0, :]
    y = points[:, 1, :]
    z = points[:, 2, :]
    xt = jnp.transpose(points, (2, 0, 1)).reshape(N, 1, 3 * B)
    fid = pl.pallas_call(
        functools.partial(_fps_body, npoint=npoint),
        out_shape=jax.ShapeDtypeStruct((npoint, 1, B), jnp.int32),
        scratch_shapes=[pltpu.VMEM((B, N), jnp.float32)],
    )(x, y, z, xt)
    return jnp.transpose(fid[:, 0, :], (1, 0))


def _gather(points, idx):
    pt = jnp.transpose(points, (0, 2, 1))
    out = jnp.take_along_axis(pt, idx[:, :, None], axis=1)
    return jnp.transpose(out, (0, 2, 1))


def _tree_argmin(vals, idxs):
    vals = list(vals)
    idxs = list(idxs)
    n = len(vals)
    gap = 1
    while gap < n:
        for i in range(0, n - gap, 2 * gap):
            lt = vals[i + gap] < vals[i]
            vals[i] = jnp.where(lt, vals[i + gap], vals[i])
            idxs[i] = jnp.where(lt, idxs[i + gap], idxs[i])
        gap *= 2
    return vals[0], idxs[0]


def _tree_add(vals):
    vals = list(vals)
    n = len(vals)
    gap = 1
    while gap < n:
        for i in range(0, n - gap, 2 * gap):
            vals[i] = vals[i] + vals[i + gap]
        gap *= 2
    return vals[0]


def _knn_group(d, K, NR, lane):
    _INF = jnp.float32(jnp.inf)
    _BIGN = jnp.int32(2**31 - 1)
    cols = [d[:, c * 128:(c + 1) * 128] for c in range(NR)]
    ncols = [lane + jnp.int32(c * 128) for c in range(NR)]
    rowid = [jnp.full((8, 128), c, jnp.int32) for c in range(NR)]

    DEPTH = 4
    work = list(cols)
    sv, sn = [], []
    for j in range(DEPTH):
        mv, mc = _tree_argmin(work, rowid)
        sv.append(mv)
        sn.append(mc * 128 + lane)
        if j < DEPTH - 1:
            work = [jnp.where(mc == c, _INF, work[c]) for c in range(NR)]

    v0, v1, v2, v3 = sv
    n0, n1, n2, n3 = sn
    outs = []
    last_v = None
    last_n = None
    for k in range(K):
        m = jnp.min(v0, axis=1, keepdims=True)
        nb = jnp.min(jnp.where(v0 == m, n0, _BIGN), axis=1, keepdims=True)
        outs.append(nb)
        last_v, last_n = m, nb
        pop = n0 == nb
        v0 = jnp.where(pop, v1, v0)
        n0 = jnp.where(pop, n1, n0)
        v1 = jnp.where(pop, v2, v1)
        n1 = jnp.where(pop, n2, n1)
        v2 = jnp.where(pop, v3, v2)
        n2 = jnp.where(pop, n3, n2)
        v3 = jnp.where(pop, _INF, v3)
        n3 = jnp.where(pop, _BIGN, n3)
    fast = jnp.concatenate(outs, axis=1)

    cnts = []
    for c in range(NR):
        less = (cols[c] < last_v) | ((cols[c] == last_v) & (ncols[c] < last_n))
        cnts.append(jnp.where(less, jnp.int32(1), jnp.int32(0)))
    total = jnp.sum(_tree_add(cnts), axis=1, keepdims=True)
    good = jnp.all(total == jnp.int32(K - 1))

    def exact():
        ik = jax.lax.broadcasted_iota(jnp.int32, (8, K), 1)

        def body(k, carry):
            outs_acc = carry[0]
            w = list(carry[1:])
            mv = w[0]
            mn = ncols[0]
            for c in range(1, NR):
                lt = (w[c] < mv) | ((w[c] == mv) & (ncols[c] < mn))
                mv = jnp.where(lt, w[c], mv)
                mn = jnp.where(lt, ncols[c], mn)
            m = jnp.min(mv, axis=1, keepdims=True)
            nb = jnp.min(jnp.where(mv == m, mn, _BIGN), axis=1, keepdims=True)
            outs_acc = outs_acc + jnp.where(ik == k, nb, jnp.int32(0))
            neww = [jnp.where(ncols[c] == nb, _INF, w[c]) for c in range(NR)]
            return (outs_acc, *neww)

        init = (jnp.zeros((8, K), jnp.int32), *cols)
        return jax.lax.fori_loop(0, K, body, init)[0]

    return jax.lax.cond(good, lambda: fast, exact)


def _knn_body(q_ref, pt_ref, qn2_ref, pn2_ref, out_ref, *, K, N, G):
    NR = N // 128
    pt = pt_ref[0]
    pn2 = pn2_ref[0]
    q = q_ref[0]
    qn2 = qn2_ref[0]
    dot = jax.lax.dot_general(q, pt, (((1,), (0,)), ((), ())),
                              preferred_element_type=jnp.float32)
    d = (qn2 + pn2) - 2.0 * dot
    lane = jax.lax.broadcasted_iota(jnp.int32, (8, 128), 1)
    res = [_knn_group(d[8 * g:8 * (g + 1), :], K, NR, lane) for g in range(G)]
    out_ref[0] = jnp.concatenate(res, axis=0)


def _knn_pallas(xyz_pl, new_xyz, nsample):
    B, _, N = xyz_pl.shape
    S = new_xyz.shape[1]
    K = nsample
    G = 4
    PT = jnp.concatenate([xyz_pl, jnp.zeros((B, 5, N), jnp.float32)], axis=1)
    Q = jnp.concatenate([new_xyz, jnp.zeros((B, S, 5), jnp.float32)], axis=-1)
    qn2 = jnp.sum(new_xyz ** 2, axis=-1)[..., None]
    pn2 = jnp.sum(jnp.transpose(xyz_pl, (0, 2, 1)) ** 2, axis=-1)[:, None, :]
    grid = (B, S // (8 * G))
    return pl.pallas_call(
        functools.partial(_knn_body, K=K, N=N, G=G),
        grid=grid,
        in_specs=[
            pl.BlockSpec((1, 8 * G, 8), lambda b, g: (b, g, 0)),
            pl.BlockSpec((1, 8, N), lambda b, g: (b, 0, 0)),
            pl.BlockSpec((1, 8 * G, 1), lambda b, g: (b, g, 0)),
            pl.BlockSpec((1, 1, N), lambda b, g: (b, 0, 0)),
        ],
        out_specs=pl.BlockSpec((1, 8 * G, K), lambda b, g: (b, g, 0)),
        out_shape=jax.ShapeDtypeStruct((B, S, K), jnp.int32),
    )(Q, PT, qn2, pn2)


def _knn(k, xyz, new_xyz):
    d = (jnp.sum(new_xyz ** 2, axis=-1)[:, :, None]
         + jnp.sum(xyz ** 2, axis=-1)[:, None, :]
         - 2.0 * jnp.einsum('bsd,bnd->bsn', new_xyz, xyz))
    _, idx = jax.lax.top_k(-d, k)
    return idx


def _group(feats, idx):
    B, C, N = feats.shape
    _, S, K = idx.shape
    ft = jnp.transpose(feats, (0, 2, 1)).reshape(B * N, C)
    flat = (idx.reshape(B, S * K) + (jnp.arange(B, dtype=jnp.int32) * N)[:, None]).reshape(-1)
    g = jnp.take(ft, flat, axis=0)
    return jnp.transpose(g.reshape(B, S, K, C), (0, 3, 1, 2))


def _conv_bn_relu(x, W, b, g, be):
    y = jnp.einsum('oc,bcsk->bosk', W, x) + b[None, :, None, None]
    mean = jnp.mean(y, axis=(0, 2, 3), keepdims=True)
    var = jnp.var(y, axis=(0, 2, 3), keepdims=True)
    y = (y - mean) / jnp.sqrt(var + 1e-5) * g[None, :, None, None] + be[None, :, None, None]
    return jax.nn.relu(y)


def _copy_kernel(x_ref, o_ref):
    o_ref[...] = x_ref[...]


def _pl_copy(x):
    return pl.pallas_call(
        _copy_kernel,
        out_shape=jax.ShapeDtypeStruct(x.shape, x.dtype),
    )(x)


def kernel(points,
           W_l1c0, b_l1c0, g_l1c0, be_l1c0,
           W_l1c1, b_l1c1, g_l1c1, be_l1c1,
           W_l1c2, b_l1c2, g_l1c2, be_l1c2,
           W_l2c0, b_l2c0, g_l2c0, be_l2c0,
           W_l2c1, b_l2c1, g_l2c1, be_l2c1,
           W_l2c2, b_l2c2, g_l2c2, be_l2c2):
    kw = locals()
    names = ["l1c0", "l1c1", "l1c2", "l2c0", "l2c1", "l2c2"]
    params = [(kw["W_" + n], kw["b_" + n], kw["g_" + n], kw["be_" + n]) for n in names]

    feats = points
    points_list, feats_list, gidx_list = [], [], []
    offs = [0, 3]
    for li in range(2):
        npoint = _KNUM_POINTS[li]
        nsample = _KNUM_SAMPLE[li]
        xyz = jnp.transpose(points, (0, 2, 1))
        fid = _fps_pallas(points, npoint)
        prop = _gather(points, fid)
        new_xyz = jnp.transpose(prop, (0, 2, 1))
        gidx = _knn_pallas(points, new_xyz, nsample)
        gp = _group(points, gidx)
        gpn = gp - prop[..., None]
        gf = gp if li == 0 else _group(feats, gidx)
        x = jnp.concatenate([gpn, gf], axis=1)
        for n in range(3):
            W, b, g, be = params[offs[li] + n]
            x = _conv_bn_relu(x, W, b, g, be)
        pf = jnp.max(x, axis=-1)
        points_list.append(prop)
        feats_list.append(pf)
        gidx_list.append(gidx)
        points = prop
        feats = pf
    return (*points_list, *feats_list, *gidx_list)


# speedup vs baseline: 2.7276x; 2.0260x over previous
import functools

import jax
import jax.numpy as jnp
from jax.experimental import pallas as pl
from jax.experimental.pallas import tpu as pltpu

_KNUM_POINTS = [1024, 256]
_KNUM_SAMPLE = [32, 64]


def _fps_body(x_ref, y_ref, z_ref, xt_ref, fid_ref, dist_ref, *, npoint):
    B, N = x_ref.shape
    dist_ref[...] = jnp.full((B, N), 1e10, jnp.float32)
    iota = jax.lax.broadcasted_iota(jnp.int32, (B, N), 1)

    def step(i, far):
        fid_ref[pl.ds(i, 1)] = far.reshape(1, 1, B)
        cx, cy, cz = [], [], []
        for b in range(B):
            fb = far[0, b]
            row = xt_ref[pl.ds(fb, 1)]
            cx.append(row[0, 0, 3 * b + 0])
            cy.append(row[0, 0, 3 * b + 1])
            cz.append(row[0, 0, 3 * b + 2])
        cxv = jnp.stack(cx).reshape(B, 1)
        cyv = jnp.stack(cy).reshape(B, 1)
        czv = jnp.stack(cz).reshape(B, 1)
        dx = x_ref[...] - cxv
        dy = y_ref[...] - cyv
        dz = z_ref[...] - czv
        d = (dx * dx + dy * dy) + dz * dz
        dist = jnp.minimum(dist_ref[...], d)
        dist_ref[...] = dist
        m = jnp.max(dist, axis=-1, keepdims=True)
        nxt = jnp.min(jnp.where(dist == m, iota, N), axis=-1, keepdims=True)
        return nxt.astype(jnp.int32).reshape(1, B)

    jax.lax.fori_loop(0, npoint, step, jnp.zeros((1, B), jnp.int32))


def _fps_pallas(points, npoint):
    B, _, N = points.shape
    x = points[:, 0, :]
    y = points[:, 1, :]
    z = points[:, 2, :]
    xt = jnp.transpose(points, (2, 0, 1)).reshape(N, 1, 3 * B)
    fid = pl.pallas_call(
        functools.partial(_fps_body, npoint=npoint),
        out_shape=jax.ShapeDtypeStruct((npoint, 1, B), jnp.int32),
        scratch_shapes=[pltpu.VMEM((B, N), jnp.float32)],
    )(x, y, z, xt)
    return jnp.transpose(fid[:, 0, :], (1, 0))


def _gather(points, idx):
    pt = jnp.transpose(points, (0, 2, 1))
    out = jnp.take_along_axis(pt, idx[:, :, None], axis=1)
    return jnp.transpose(out, (0, 2, 1))


def _tree_argmin(vals, idxs):
    vals = list(vals)
    idxs = list(idxs)
    n = len(vals)
    gap = 1
    while gap < n:
        for i in range(0, n - gap, 2 * gap):
            lt = vals[i + gap] < vals[i]
            vals[i] = jnp.where(lt, vals[i + gap], vals[i])
            idxs[i] = jnp.where(lt, idxs[i + gap], idxs[i])
        gap *= 2
    return vals[0], idxs[0]


def _tree_add(vals):
    vals = list(vals)
    n = len(vals)
    gap = 1
    while gap < n:
        for i in range(0, n - gap, 2 * gap):
            vals[i] = vals[i] + vals[i + gap]
        gap *= 2
    return vals[0]


def _knn_group(d, K, NR, lane):
    R = d.shape[0]
    _INF = jnp.float32(jnp.inf)
    _BIGN = jnp.int32(2**31 - 1)
    cols = [d[:, c * 128:(c + 1) * 128] for c in range(NR)]
    ncols = [lane + jnp.int32(c * 128) for c in range(NR)]
    rowid = [jnp.full((R, 128), c, jnp.int32) for c in range(NR)]

    DEPTH = 4
    work = list(cols)
    sv, sn = [], []
    for j in range(DEPTH):
        mv, mc = _tree_argmin(work, rowid)
        sv.append(mv)
        sn.append(mc * 128 + lane)
        if j < DEPTH - 1:
            work = [jnp.where(mc == c, _INF, work[c]) for c in range(NR)]

    v0, v1, v2, v3 = sv
    n0, n1, n2, n3 = sn
    outs = []
    last_v = None
    last_n = None
    for k in range(K):
        m = jnp.min(v0, axis=1, keepdims=True)
        nb = jnp.min(jnp.where(v0 == m, n0, _BIGN), axis=1, keepdims=True)
        outs.append(nb)
        last_v, last_n = m, nb
        pop = n0 == nb
        v0 = jnp.where(pop, v1, v0)
        n0 = jnp.where(pop, n1, n0)
        v1 = jnp.where(pop, v2, v1)
        n1 = jnp.where(pop, n2, n1)
        v2 = jnp.where(pop, v3, v2)
        n2 = jnp.where(pop, n3, n2)
        v3 = jnp.where(pop, _INF, v3)
        n3 = jnp.where(pop, _BIGN, n3)
    fast = jnp.concatenate(outs, axis=1)

    cnts = []
    for c in range(NR):
        less = (cols[c] < last_v) | ((cols[c] == last_v) & (ncols[c] < last_n))
        cnts.append(jnp.where(less, jnp.int32(1), jnp.int32(0)))
    total = jnp.sum(_tree_add(cnts), axis=1, keepdims=True)
    good = jnp.all(total == jnp.int32(K - 1))

    def exact():
        ik = jax.lax.broadcasted_iota(jnp.int32, (R, K), 1)

        def body(k, carry):
            outs_acc = carry[0]
            w = list(carry[1:])
            mv = w[0]
            mn = ncols[0]
            for c in range(1, NR):
                lt = (w[c] < mv) | ((w[c] == mv) & (ncols[c] < mn))
                mv = jnp.where(lt, w[c], mv)
                mn = jnp.where(lt, ncols[c], mn)
            m = jnp.min(mv, axis=1, keepdims=True)
            nb = jnp.min(jnp.where(mv == m, mn, _BIGN), axis=1, keepdims=True)
            outs_acc = outs_acc + jnp.where(ik == k, nb, jnp.int32(0))
            neww = [jnp.where(ncols[c] == nb, _INF, w[c]) for c in range(NR)]
            return (outs_acc, *neww)

        init = (jnp.zeros((R, K), jnp.int32), *cols)
        return jax.lax.fori_loop(0, K, body, init)[0]

    return jax.lax.cond(good, lambda: fast, exact)


def _knn_body(q_ref, pt_ref, qn2_ref, pn2_ref, out_ref, *, K, N, G):
    NR = N // 128
    pt = pt_ref[0]
    pn2 = pn2_ref[0]
    q = q_ref[0]
    qn2 = qn2_ref[0]
    dot = jax.lax.dot_general(q, pt, (((1,), (0,)), ((), ())),
                              preferred_element_type=jnp.float32)
    d = (qn2 + pn2) - 2.0 * dot
    lane = jax.lax.broadcasted_iota(jnp.int32, (8 * G, 128), 1)
    out_ref[0] = _knn_group(d, K, NR, lane)


def _knn_pallas(xyz_pl, new_xyz, nsample):
    B, _, N = xyz_pl.shape
    S = new_xyz.shape[1]
    K = nsample
    G = 4
    PT = jnp.concatenate([xyz_pl, jnp.zeros((B, 5, N), jnp.float32)], axis=1)
    Q = jnp.concatenate([new_xyz, jnp.zeros((B, S, 5), jnp.float32)], axis=-1)
    qn2 = jnp.sum(new_xyz ** 2, axis=-1)[..., None]
    pn2 = jnp.sum(jnp.transpose(xyz_pl, (0, 2, 1)) ** 2, axis=-1)[:, None, :]
    grid = (B, S // (8 * G))
    return pl.pallas_call(
        functools.partial(_knn_body, K=K, N=N, G=G),
        grid=grid,
        in_specs=[
            pl.BlockSpec((1, 8 * G, 8), lambda b, g: (b, g, 0)),
            pl.BlockSpec((1, 8, N), lambda b, g: (b, 0, 0)),
            pl.BlockSpec((1, 8 * G, 1), lambda b, g: (b, g, 0)),
            pl.BlockSpec((1, 1, N), lambda b, g: (b, 0, 0)),
        ],
        out_specs=pl.BlockSpec((1, 8 * G, K), lambda b, g: (b, g, 0)),
        out_shape=jax.ShapeDtypeStruct((B, S, K), jnp.int32),
    )(Q, PT, qn2, pn2)


def _knn(k, xyz, new_xyz):
    d = (jnp.sum(new_xyz ** 2, axis=-1)[:, :, None]
         + jnp.sum(xyz ** 2, axis=-1)[:, None, :]
         - 2.0 * jnp.einsum('bsd,bnd->bsn', new_xyz, xyz))
    _, idx = jax.lax.top_k(-d, k)
    return idx


def _group(feats, idx):
    B, C, N = feats.shape
    _, S, K = idx.shape
    ft = jnp.transpose(feats, (0, 2, 1)).reshape(B * N, C)
    flat = (idx.reshape(B, S * K) + (jnp.arange(B, dtype=jnp.int32) * N)[:, None]).reshape(-1)
    g = jnp.take(ft, flat, axis=0)
    return jnp.transpose(g.reshape(B, S, K, C), (0, 3, 1, 2))


def _conv_bn_relu(x, W, b, g, be):
    y = jnp.einsum('oc,bcsk->bosk', W, x) + b[None, :, None, None]
    mean = jnp.mean(y, axis=(0, 2, 3), keepdims=True)
    var = jnp.var(y, axis=(0, 2, 3), keepdims=True)
    y = (y - mean) / jnp.sqrt(var + 1e-5) * g[None, :, None, None] + be[None, :, None, None]
    return jax.nn.relu(y)


def _copy_kernel(x_ref, o_ref):
    o_ref[...] = x_ref[...]


def _pl_copy(x):
    return pl.pallas_call(
        _copy_kernel,
        out_shape=jax.ShapeDtypeStruct(x.shape, x.dtype),
    )(x)


def kernel(points,
           W_l1c0, b_l1c0, g_l1c0, be_l1c0,
           W_l1c1, b_l1c1, g_l1c1, be_l1c1,
           W_l1c2, b_l1c2, g_l1c2, be_l1c2,
           W_l2c0, b_l2c0, g_l2c0, be_l2c0,
           W_l2c1, b_l2c1, g_l2c1, be_l2c1,
           W_l2c2, b_l2c2, g_l2c2, be_l2c2):
    kw = locals()
    names = ["l1c0", "l1c1", "l1c2", "l2c0", "l2c1", "l2c2"]
    params = [(kw["W_" + n], kw["b_" + n], kw["g_" + n], kw["be_" + n]) for n in names]

    feats = points
    points_list, feats_list, gidx_list = [], [], []
    offs = [0, 3]
    for li in range(2):
        npoint = _KNUM_POINTS[li]
        nsample = _KNUM_SAMPLE[li]
        xyz = jnp.transpose(points, (0, 2, 1))
        fid = _fps_pallas(points, npoint)
        prop = _gather(points, fid)
        new_xyz = jnp.transpose(prop, (0, 2, 1))
        gidx = _knn_pallas(points, new_xyz, nsample)
        gp = _group(points, gidx)
        gpn = gp - prop[..., None]
        gf = gp if li == 0 else _group(feats, gidx)
        x = jnp.concatenate([gpn, gf], axis=1)
        for n in range(3):
            W, b, g, be = params[offs[li] + n]
            x = _conv_bn_relu(x, W, b, g, be)
        pf = jnp.max(x, axis=-1)
        points_list.append(prop)
        feats_list.append(pf)
        gidx_list.append(gidx)
        points = prop
        feats = pf
    return (*points_list, *feats_list, *gidx_list)


# speedup vs baseline: 3.2550x; 1.1933x over previous
import functools

import jax
import jax.numpy as jnp
from jax.experimental import pallas as pl
from jax.experimental.pallas import tpu as pltpu

_KNUM_POINTS = [1024, 256]
_KNUM_SAMPLE = [32, 64]


def _fps_body(x_ref, y_ref, z_ref, xt_ref, fid_ref, dist_ref, *, npoint):
    B, N = x_ref.shape
    dist_ref[...] = jnp.full((B, N), 1e10, jnp.float32)
    iota = jax.lax.broadcasted_iota(jnp.int32, (B, N), 1)

    def step(i, far):
        fid_ref[pl.ds(i, 1)] = far.reshape(1, 1, B)
        cx, cy, cz = [], [], []
        for b in range(B):
            fb = far[0, b]
            row = xt_ref[pl.ds(fb, 1)]
            cx.append(row[0, 0, 3 * b + 0])
            cy.append(row[0, 0, 3 * b + 1])
            cz.append(row[0, 0, 3 * b + 2])
        cxv = jnp.stack(cx).reshape(B, 1)
        cyv = jnp.stack(cy).reshape(B, 1)
        czv = jnp.stack(cz).reshape(B, 1)
        dx = x_ref[...] - cxv
        dy = y_ref[...] - cyv
        dz = z_ref[...] - czv
        d = (dx * dx + dy * dy) + dz * dz
        dist = jnp.minimum(dist_ref[...], d)
        dist_ref[...] = dist
        m = jnp.max(dist, axis=-1, keepdims=True)
        nxt = jnp.min(jnp.where(dist == m, iota, N), axis=-1, keepdims=True)
        return nxt.astype(jnp.int32).reshape(1, B)

    jax.lax.fori_loop(0, npoint, step, jnp.zeros((1, B), jnp.int32))


def _fps_pallas(points, npoint):
    B, _, N = points.shape
    x = points[:, 0, :]
    y = points[:, 1, :]
    z = points[:, 2, :]
    xt = jnp.transpose(points, (2, 0, 1)).reshape(N, 1, 3 * B)
    fid = pl.pallas_call(
        functools.partial(_fps_body, npoint=npoint),
        out_shape=jax.ShapeDtypeStruct((npoint, 1, B), jnp.int32),
        scratch_shapes=[pltpu.VMEM((B, N), jnp.float32)],
    )(x, y, z, xt)
    return jnp.transpose(fid[:, 0, :], (1, 0))


def _gather(points, idx):
    pt = jnp.transpose(points, (0, 2, 1))
    out = jnp.take_along_axis(pt, idx[:, :, None], axis=1)
    return jnp.transpose(out, (0, 2, 1))


def _tree_argmin(vals, idxs):
    vals = list(vals)
    idxs = list(idxs)
    n = len(vals)
    gap = 1
    while gap < n:
        for i in range(0, n - gap, 2 * gap):
            lt = vals[i + gap] < vals[i]
            vals[i] = jnp.where(lt, vals[i + gap], vals[i])
            idxs[i] = jnp.where(lt, idxs[i + gap], idxs[i])
        gap *= 2
    return vals[0], idxs[0]


def _tree_add(vals):
    vals = list(vals)
    n = len(vals)
    gap = 1
    while gap < n:
        for i in range(0, n - gap, 2 * gap):
            vals[i] = vals[i] + vals[i + gap]
        gap *= 2
    return vals[0]


def _knn_group(d, K, NR, lane):
    R = d.shape[0]
    _INF = jnp.float32(jnp.inf)
    _BIGN = jnp.int32(2**31 - 1)
    cols = [d[:, c * 128:(c + 1) * 128] for c in range(NR)]
    ncols = [lane + jnp.int32(c * 128) for c in range(NR)]
    rowid = [jnp.full((R, 128), c, jnp.int32) for c in range(NR)]

    DEPTH = 4
    work = list(cols)
    sv, sn = [], []
    for j in range(DEPTH):
        mv, mc = _tree_argmin(work, rowid)
        sv.append(mv)
        sn.append(mc * 128 + lane)
        if j < DEPTH - 1:
            work = [jnp.where(mc == c, _INF, work[c]) for c in range(NR)]

    v0, v1, v2, v3 = sv
    n0, n1, n2, n3 = sn
    outs = []
    last_v = None
    last_n = None
    for k in range(K):
        m = jnp.min(v0, axis=1, keepdims=True)
        nb = jnp.min(jnp.where(v0 == m, n0, _BIGN), axis=1, keepdims=True)
        outs.append(nb)
        last_v, last_n = m, nb
        pop = n0 == nb
        v0 = jnp.where(pop, v1, v0)
        n0 = jnp.where(pop, n1, n0)
        v1 = jnp.where(pop, v2, v1)
        n1 = jnp.where(pop, n2, n1)
        v2 = jnp.where(pop, v3, v2)
        n2 = jnp.where(pop, n3, n2)
        v3 = jnp.where(pop, _INF, v3)
        n3 = jnp.where(pop, _BIGN, n3)
    fast = jnp.concatenate(outs, axis=1)

    cnts = []
    for c in range(NR):
        less = (cols[c] < last_v) | ((cols[c] == last_v) & (ncols[c] < last_n))
        cnts.append(jnp.where(less, jnp.int32(1), jnp.int32(0)))
    total = jnp.sum(_tree_add(cnts), axis=1, keepdims=True)
    good = jnp.all(total == jnp.int32(K - 1))

    def exact():
        ik = jax.lax.broadcasted_iota(jnp.int32, (R, K), 1)

        def body(k, carry):
            outs_acc = carry[0]
            w = list(carry[1:])
            mv = w[0]
            mn = ncols[0]
            for c in range(1, NR):
                lt = (w[c] < mv) | ((w[c] == mv) & (ncols[c] < mn))
                mv = jnp.where(lt, w[c], mv)
                mn = jnp.where(lt, ncols[c], mn)
            m = jnp.min(mv, axis=1, keepdims=True)
            nb = jnp.min(jnp.where(mv == m, mn, _BIGN), axis=1, keepdims=True)
            outs_acc = outs_acc + jnp.where(ik == k, nb, jnp.int32(0))
            neww = [jnp.where(ncols[c] == nb, _INF, w[c]) for c in range(NR)]
            return (outs_acc, *neww)

        init = (jnp.zeros((R, K), jnp.int32), *cols)
        return jax.lax.fori_loop(0, K, body, init)[0]

    return jax.lax.cond(good, lambda: fast, exact)


def _knn_body(q_ref, pt_ref, qn2_ref, pn2_ref, out_ref, *, K, N, G):
    NR = N // 128
    pt = pt_ref[0]
    pn2 = pn2_ref[0]
    q = q_ref[0]
    qn2 = qn2_ref[0]
    dot = jax.lax.dot_general(q, pt, (((1,), (0,)), ((), ())),
                              preferred_element_type=jnp.float32)
    d = (qn2 + pn2) - 2.0 * dot
    lane = jax.lax.broadcasted_iota(jnp.int32, (8 * G, 128), 1)
    out_ref[0] = _knn_group(d, K, NR, lane)


def _knn_pallas(xyz_pl, new_xyz, nsample):
    B, _, N = xyz_pl.shape
    S = new_xyz.shape[1]
    K = nsample
    G = 8
    PT = jnp.concatenate([xyz_pl, jnp.zeros((B, 5, N), jnp.float32)], axis=1)
    Q = jnp.concatenate([new_xyz, jnp.zeros((B, S, 5), jnp.float32)], axis=-1)
    qn2 = jnp.sum(new_xyz ** 2, axis=-1)[..., None]
    pn2 = jnp.sum(jnp.transpose(xyz_pl, (0, 2, 1)) ** 2, axis=-1)[:, None, :]
    grid = (B, S // (8 * G))
    return pl.pallas_call(
        functools.partial(_knn_body, K=K, N=N, G=G),
        grid=grid,
        in_specs=[
            pl.BlockSpec((1, 8 * G, 8), lambda b, g: (b, g, 0)),
            pl.BlockSpec((1, 8, N), lambda b, g: (b, 0, 0)),
            pl.BlockSpec((1, 8 * G, 1), lambda b, g: (b, g, 0)),
            pl.BlockSpec((1, 1, N), lambda b, g: (b, 0, 0)),
        ],
        out_specs=pl.BlockSpec((1, 8 * G, K), lambda b, g: (b, g, 0)),
        out_shape=jax.ShapeDtypeStruct((B, S, K), jnp.int32),
    )(Q, PT, qn2, pn2)


def _knn(k, xyz, new_xyz):
    d = (jnp.sum(new_xyz ** 2, axis=-1)[:, :, None]
         + jnp.sum(xyz ** 2, axis=-1)[:, None, :]
         - 2.0 * jnp.einsum('bsd,bnd->bsn', new_xyz, xyz))
    _, idx = jax.lax.top_k(-d, k)
    return idx


def _group(feats, idx):
    B, C, N = feats.shape
    _, S, K = idx.shape
    ft = jnp.transpose(feats, (0, 2, 1)).reshape(B * N, C)
    flat = (idx.reshape(B, S * K) + (jnp.arange(B, dtype=jnp.int32) * N)[:, None]).reshape(-1)
    g = jnp.take(ft, flat, axis=0)
    return jnp.transpose(g.reshape(B, S, K, C), (0, 3, 1, 2))


def _conv_bn_relu(x, W, b, g, be):
    y = jnp.einsum('oc,bcsk->bosk', W, x) + b[None, :, None, None]
    mean = jnp.mean(y, axis=(0, 2, 3), keepdims=True)
    var = jnp.var(y, axis=(0, 2, 3), keepdims=True)
    y = (y - mean) / jnp.sqrt(var + 1e-5) * g[None, :, None, None] + be[None, :, None, None]
    return jax.nn.relu(y)


def _copy_kernel(x_ref, o_ref):
    o_ref[...] = x_ref[...]


def _pl_copy(x):
    return pl.pallas_call(
        _copy_kernel,
        out_shape=jax.ShapeDtypeStruct(x.shape, x.dtype),
    )(x)


def kernel(points,
           W_l1c0, b_l1c0, g_l1c0, be_l1c0,
           W_l1c1, b_l1c1, g_l1c1, be_l1c1,
           W_l1c2, b_l1c2, g_l1c2, be_l1c2,
           W_l2c0, b_l2c0, g_l2c0, be_l2c0,
           W_l2c1, b_l2c1, g_l2c1, be_l2c1,
           W_l2c2, b_l2c2, g_l2c2, be_l2c2):
    kw = locals()
    names = ["l1c0", "l1c1", "l1c2", "l2c0", "l2c1", "l2c2"]
    params = [(kw["W_" + n], kw["b_" + n], kw["g_" + n], kw["be_" + n]) for n in names]

    feats = points
    points_list, feats_list, gidx_list = [], [], []
    offs = [0, 3]
    for li in range(2):
        npoint = _KNUM_POINTS[li]
        nsample = _KNUM_SAMPLE[li]
        xyz = jnp.transpose(points, (0, 2, 1))
        fid = _fps_pallas(points, npoint)
        prop = _gather(points, fid)
        new_xyz = jnp.transpose(prop, (0, 2, 1))
        gidx = _knn_pallas(points, new_xyz, nsample)
        gp = _group(points, gidx)
        gpn = gp - prop[..., None]
        gf = gp if li == 0 else _group(feats, gidx)
        x = jnp.concatenate([gpn, gf], axis=1)
        for n in range(3):
            W, b, g, be = params[offs[li] + n]
            x = _conv_bn_relu(x, W, b, g, be)
        pf = jnp.max(x, axis=-1)
        points_list.append(prop)
        feats_list.append(pf)
        gidx_list.append(gidx)
        points = prop
        feats = pf
    return (*points_list, *feats_list, *gidx_list)
